# baseline (speedup 1.0000x reference)
import jax
import jax.numpy as jnp
from jax import lax
from jax.experimental import pallas as pl
from jax.experimental.pallas import tpu as pltpu
from jax.experimental.pallas import tpu_sc as plsc

N = 10000
E = 160000
IN_DIM = 256
HD = 512
DH = 256
ALPHA = 0.2

NC = 2
NS = 16
NW = NC * NS
L = 16
G = 32

NP = 10240
NSEG = 2 * NP

EW = 5024
EW_LAST = E - (NW - 1) * EW
ES = E // NS
WIN = 2000
NWIN = ES // WIN

Q = 1250
QR = 1280
NQ = 4

_f32 = jnp.float32
_i32 = jnp.int32

_SC_PARAMS = pltpu.CompilerParams(
    use_tc_tiling_on_sc=False, needs_layout_passes=False)


def _mm_body(x_ref, ws_ref, wd_ref, wv_ref, hs_ref, hd_ref, hv_ref):
    xb = x_ref[...]
    hs_ref[...] = jnp.dot(xb, ws_ref[...], preferred_element_type=_f32)
    hd_ref[...] = jnp.dot(xb, wd_ref[...], preferred_element_type=_f32)
    hv_ref[...] = jnp.dot(xb, wv_ref[...], preferred_element_type=_f32)


def _projections(x, w_s, w_d, w_v):
    blk = 1000
    return pl.pallas_call(
        _mm_body,
        grid=(N // blk,),
        in_specs=[
            pl.BlockSpec((blk, IN_DIM), lambda i: (i, 0)),
            pl.BlockSpec((IN_DIM, HD), lambda i: (0, 0)),
            pl.BlockSpec((IN_DIM, HD), lambda i: (0, 0)),
            pl.BlockSpec((IN_DIM, HD), lambda i: (0, 0)),
        ],
        out_specs=[
            pl.BlockSpec((blk, HD), lambda i: (i, 0)),
            pl.BlockSpec((blk, HD), lambda i: (i, 0)),
            pl.BlockSpec((blk, HD), lambda i: (i, 0)),
        ],
        out_shape=[
            jax.ShapeDtypeStruct((N, HD), _f32),
            jax.ShapeDtypeStruct((N, HD), _f32),
            jax.ShapeDtypeStruct((N, HD), _f32),
        ],
    )(x, w_s, w_d, w_v)


def _k2_body(hs_hbm, hd_hbm, src_hbm, dst_hbm, av_hbm,
             p0_hbm, p1_hbm, s0_hbm, s1_hbm,
             srca, dsta, avv, p0a, p1a, s0a, s1a,
             hsA, hdA, hsB, hdB, semA1, semA2, semB1, semB2):
    c = lax.axis_index("c")
    s = lax.axis_index("s")
    wid = s * NC + c
    base = wid * EW
    last = wid == NW - 1
    ng = jnp.where(last, EW_LAST // G, EW // G)

    pltpu.sync_copy(av_hbm, avv)

    @pl.when(jnp.logical_not(last))
    def _():
        pltpu.sync_copy(src_hbm.at[pl.ds(base, EW)], srca.at[pl.ds(0, EW)])
        pltpu.sync_copy(dst_hbm.at[pl.ds(base, EW)], dsta.at[pl.ds(0, EW)])

    @pl.when(last)
    def _():
        pltpu.sync_copy(src_hbm.at[pl.ds(base, EW_LAST)],
                        srca.at[pl.ds(0, EW_LAST)])
        pltpu.sync_copy(dst_hbm.at[pl.ds(base, EW_LAST)],
                        dsta.at[pl.ds(0, EW_LAST)])

    iota = lax.iota(_i32, L)

    def issue(hsX, hdX, semX1, semX2, k):
        b = k * G
        pltpu.async_copy(hs_hbm.at[srca.at[pl.ds(b, G)]], hsX, semX1)
        pltpu.async_copy(hd_hbm.at[dsta.at[pl.ds(b, G)]], hdX, semX2)

    def wait(hsX, hdX, semX1, semX2):
        pltpu.make_async_copy(
            hs_hbm.at[srca.at[pl.ds(0, G)]], hsX, semX1).wait()
        pltpu.make_async_copy(
            hd_hbm.at[dsta.at[pl.ds(0, G)]], hdX, semX2).wait()

    def do_group(hsX, hdX, k):
        b = k * G
        for h in (0, 1):
            rid = iota + L * h

            def dloop(j, accs):
                acc0, acc1 = accs
                for u in range(2):
                    d = j * 2 + u
                    df = ((d + iota) & (L - 1)) + (d & ~(L - 1))
                    z0 = (plsc.load_gather(hsX, [rid, df])
                          + plsc.load_gather(hdX, [rid, df]))
                    l0 = jnp.maximum(z0, 0.0) + ALPHA * jnp.minimum(z0, 0.0)
                    acc0 = acc0 + l0 * plsc.load_gather(avv, [df])
                    df1 = df + DH
                    z1 = (plsc.load_gather(hsX, [rid, df1])
                          + plsc.load_gather(hdX, [rid, df1]))
                    l1 = jnp.maximum(z1, 0.0) + ALPHA * jnp.minimum(z1, 0.0)
                    acc1 = acc1 + l1 * plsc.load_gather(avv, [df1])
                return (acc0, acc1)

            acc0, acc1 = lax.fori_loop(
                0, DH // 2, dloop,
                (jnp.zeros((L,), _f32), jnp.zeros((L,), _f32)))
            o = b + L * h
            s0a[pl.ds(o, L)] = acc0
            s1a[pl.ds(o, L)] = acc1
            p0a[pl.ds(o, L)] = jnp.exp(acc0)
            p1a[pl.ds(o, L)] = jnp.exp(acc1)

    issue(hsA, hdA, semA1, semA2, 0)

    def piter(m, _):
        ka = 2 * m
        kb = ka + 1

        @pl.when(kb < ng)
        def _():
            issue(hsB, hdB, semB1, semB2, kb)

        wait(hsA, hdA, semA1, semA2)
        do_group(hsA, hdA, ka)

        @pl.when(ka + 2 < ng)
        def _():
            issue(hsA, hdA, semA1, semA2, ka + 2)

        @pl.when(kb < ng)
        def _():
            wait(hsB, hdB, semB1, semB2)
            do_group(hsB, hdB, kb)
        return 0

    lax.fori_loop(0, (ng + 1) // 2, piter, 0)

    @pl.when(jnp.logical_not(last))
    def _():
        pltpu.sync_copy(p0a.at[pl.ds(0, EW)], p0_hbm.at[pl.ds(base, EW)])
        pltpu.sync_copy(p1a.at[pl.ds(0, EW)], p1_hbm.at[pl.ds(base, EW)])
        pltpu.sync_copy(s0a.at[pl.ds(0, EW)], s0_hbm.at[pl.ds(base, EW)])
        pltpu.sync_copy(s1a.at[pl.ds(0, EW)], s1_hbm.at[pl.ds(base, EW)])

    @pl.when(last)
    def _():
        pltpu.sync_copy(p0a.at[pl.ds(0, EW_LAST)],
                        p0_hbm.at[pl.ds(base, EW_LAST)])
        pltpu.sync_copy(p1a.at[pl.ds(0, EW_LAST)],
                        p1_hbm.at[pl.ds(base, EW_LAST)])
        pltpu.sync_copy(s0a.at[pl.ds(0, EW_LAST)],
                        s0_hbm.at[pl.ds(base, EW_LAST)])
        pltpu.sync_copy(s1a.at[pl.ds(0, EW_LAST)],
                        s1_hbm.at[pl.ds(base, EW_LAST)])


def _edge_scores(h_s, h_d, src, dst, av):
    mesh = plsc.VectorSubcoreMesh(core_axis_name="c", subcore_axis_name="s")
    fn = pl.kernel(
        _k2_body,
        out_type=(
            jax.ShapeDtypeStruct((E,), _f32),
            jax.ShapeDtypeStruct((E,), _f32),
            jax.ShapeDtypeStruct((E,), _f32),
            jax.ShapeDtypeStruct((E,), _f32),
        ),
        mesh=mesh,
        compiler_params=_SC_PARAMS,
        scratch_types=[
            pltpu.VMEM((EW,), _i32),
            pltpu.VMEM((EW,), _i32),
            pltpu.VMEM((HD,), _f32),
            pltpu.VMEM((EW,), _f32),
            pltpu.VMEM((EW,), _f32),
            pltpu.VMEM((EW,), _f32),
            pltpu.VMEM((EW,), _f32),
            pltpu.VMEM((G, HD), _f32),
            pltpu.VMEM((G, HD), _f32),
            pltpu.VMEM((G, HD), _f32),
            pltpu.VMEM((G, HD), _f32),
            pltpu.SemaphoreType.DMA,
            pltpu.SemaphoreType.DMA,
            pltpu.SemaphoreType.DMA,
            pltpu.SemaphoreType.DMA,
        ],
    )
    return fn(h_s, h_d, src, dst, av)


def _kseg_body(dst_hbm, p0_hbm, p1_hbm, s0_hbm, s1_hbm,
               psum_hbm, pmax_hbm,
               dsta, p0a, p1a, s0a, s1a, ssum, smax, widt):
    c = lax.axis_index("c")
    s = lax.axis_index("s")
    wid = s * NC + c
    base = wid * EW
    last = wid == NW - 1
    ng = jnp.where(last, EW_LAST // L, EW // L)

    @pl.when(jnp.logical_not(last))
    def _():
        pltpu.sync_copy(dst_hbm.at[pl.ds(base, EW)], dsta.at[pl.ds(0, EW)])
        pltpu.sync_copy(p0_hbm.at[pl.ds(base, EW)], p0a.at[pl.ds(0, EW)])
        pltpu.sync_copy(p1_hbm.at[pl.ds(base, EW)], p1a.at[pl.ds(0, EW)])
        pltpu.sync_copy(s0_hbm.at[pl.ds(base, EW)], s0a.at[pl.ds(0, EW)])
        pltpu.sync_copy(s1_hbm.at[pl.ds(base, EW)], s1a.at[pl.ds(0, EW)])

    @pl.when(last)
    def _():
        pltpu.sync_copy(dst_hbm.at[pl.ds(base, EW_LAST)],
                        dsta.at[pl.ds(0, EW_LAST)])
        pltpu.sync_copy(p0_hbm.at[pl.ds(base, EW_LAST)],
                        p0a.at[pl.ds(0, EW_LAST)])
        pltpu.sync_copy(p1_hbm.at[pl.ds(base, EW_LAST)],
                        p1a.at[pl.ds(0, EW_LAST)])
        pltpu.sync_copy(s0_hbm.at[pl.ds(base, EW_LAST)],
                        s0a.at[pl.ds(0, EW_LAST)])
        pltpu.sync_copy(s1_hbm.at[pl.ds(base, EW_LAST)],
                        s1a.at[pl.ds(0, EW_LAST)])

    def init_body(i, _):
        ssum[pl.ds(i * L, L)] = jnp.zeros((L,), _f32)
        smax[pl.ds(i * L, L)] = jnp.full((L,), -1e30, _f32)
        return 0
    lax.fori_loop(0, NSEG // L, init_body, 0)

    iota = lax.iota(_i32, L)

    def group(g, _):
        b = g * L
        dv = dsta[pl.ds(b, L)]
        dv1 = dv + NP
        sc0 = s0a[pl.ds(b, L)]
        sc1 = s1a[pl.ds(b, L)]
        p0v = p0a[pl.ds(b, L)]
        p1v = p1a[pl.ds(b, L)]

        def upd_cond(rem):
            return jnp.any(rem)

        def upd_body(rem):
            plsc.store_scatter(widt, [dv], iota, mask=rem)
            win = rem & (plsc.load_gather(widt, [dv]) == iota)
            plsc.store_scatter(
                smax, [dv],
                jnp.maximum(plsc.load_gather(smax, [dv]), sc0), mask=win)
            plsc.store_scatter(
                smax, [dv1],
                jnp.maximum(plsc.load_gather(smax, [dv1]), sc1), mask=win)
            plsc.store_scatter(
                ssum, [dv], plsc.load_gather(ssum, [dv]) + p0v, mask=win)
            plsc.store_scatter(
                ssum, [dv1], plsc.load_gather(ssum, [dv1]) + p1v, mask=win)
            return rem & jnp.logical_not(win)

        lax.while_loop(upd_cond, upd_body, jnp.full((L,), True, jnp.bool_))
        return 0

    lax.fori_loop(0, ng, group, 0)
    pltpu.sync_copy(ssum, psum_hbm.at[pl.ds(wid * NSEG, NSEG)])
    pltpu.sync_copy(smax, pmax_hbm.at[pl.ds(wid * NSEG, NSEG)])


def _segment_tables(dst, p0, p1, s0, s1):
    mesh = plsc.VectorSubcoreMesh(core_axis_name="c", subcore_axis_name="s")
    fn = pl.kernel(
        _kseg_body,
        out_type=(
            jax.ShapeDtypeStruct((NW * NSEG,), _f32),
            jax.ShapeDtypeStruct((NW * NSEG,), _f32),
        ),
        mesh=mesh,
        compiler_params=_SC_PARAMS,
        scratch_types=[
            pltpu.VMEM((EW,), _i32),
            pltpu.VMEM((EW,), _f32),
            pltpu.VMEM((EW,), _f32),
            pltpu.VMEM((EW,), _f32),
            pltpu.VMEM((EW,), _f32),
            pltpu.VMEM((NSEG,), _f32),
            pltpu.VMEM((NSEG,), _f32),
            pltpu.VMEM((NP,), _i32),
        ],
    )
    return fn(dst, p0, p1, s0, s1)


def _k2b_body(ps_ref, pm_ref, den_ref):
    ssum = jnp.sum(ps_ref[...], axis=0, keepdims=True)
    smax = jnp.max(pm_ref[...], axis=0, keepdims=True)
    den_ref[...] = ssum + 1e-9 * jnp.exp(smax)


def _denominator(psum, pmax):
    return pl.pallas_call(
        _k2b_body,
        out_shape=jax.ShapeDtypeStruct((1, NSEG), _f32),
    )(psum.reshape(NW, NSEG), pmax.reshape(NW, NSEG))


def _k3_body(hv_hbm, src_hbm, dst_hbm, p0_hbm, p1_hbm, den_hbm, zer_hbm,
             out_hbm, attn_hbm,
             denv, sidw, didw, p0w, p1w, attw,
             psrc, pdl, pw0, pw1, rowsA, rowsB, acc,
             semA, semB, semSA, semSB):
    c = lax.axis_index("c")
    s = lax.axis_index("s")
    ebase = s * ES
    iota = lax.iota(_i32, L)

    pltpu.sync_copy(den_hbm, denv)

    def initp(i, _):
        psrc[pl.ds(i * L, L)] = jnp.zeros((L,), _i32)
        return 0
    lax.fori_loop(0, (WIN + 2 * L) // L, initp, 0)

    def issue(rowsX, semX, b):
        pltpu.async_copy(hv_hbm.at[psrc.at[pl.ds(b, G)]], rowsX, semX)

    def wait(rowsX, semX):
        pltpu.make_async_copy(
            hv_hbm.at[psrc.at[pl.ds(0, G)]], rowsX, semX).wait()

    def do_wloop(rowsX, k):
        b = k * G
        for h in (0, 1):
            rid = iota + L * h
            w0 = pw0[pl.ds(b + L * h, L)]
            w1 = pw1[pl.ds(b + L * h, L)]

            def wloop(j, _):
                for u in range(2):
                    d = j * 2 + u
                    df = ((d + iota) & (L - 1)) + (d & ~(L - 1))
                    plsc.store_scatter(
                        rowsX, [rid, df],
                        plsc.load_gather(rowsX, [rid, df]) * w0)
                    df1 = df + DH
                    plsc.store_scatter(
                        rowsX, [rid, df1],
                        plsc.load_gather(rowsX, [rid, df1]) * w1)
                return 0
            lax.fori_loop(0, DH // 2, wloop, 0)

    def issue_s(rowsX, semSX, k):
        b = k * G
        dlv0 = pdl[pl.ds(b, L)]
        dlv1 = pdl[pl.ds(b + L, L)]
        pltpu.async_copy(rowsX.at[pl.ds(0, L)], acc.at[dlv0], semSX,
                         add=True)
        pltpu.async_copy(rowsX.at[pl.ds(L, L)], acc.at[dlv1], semSX,
                         add=True)

    def wait_s(rowsX, semSX):
        pltpu.make_async_copy(
            rowsX.at[pl.ds(0, L)], acc.at[iota], semSX).wait()
        pltpu.make_async_copy(
            rowsX.at[pl.ds(L, L)], acc.at[iota], semSX).wait()

    def slab(q, _):
        qbase = c * (NQ * Q) + q * Q

        pltpu.sync_copy(zer_hbm, acc.at[pl.ds(s * (QR // NS), QR // NS)])
        plsc.subcore_barrier()

        def window(w, _):
            w0e = ebase + w * WIN
            pltpu.sync_copy(src_hbm.at[pl.ds(w0e, WIN)], sidw)
            pltpu.sync_copy(dst_hbm.at[pl.ds(w0e, WIN)], didw)
            pltpu.sync_copy(p0_hbm.at[pl.ds(w0e, WIN)], p0w)
            pltpu.sync_copy(p1_hbm.at[pl.ds(w0e, WIN)], p1w)

            def scan(g, pc):
                b = g * L
                dv = didw[pl.ds(b, L)]
                a0 = p0w[pl.ds(b, L)] / plsc.load_gather(denv, [dv])
                a1 = p1w[pl.ds(b, L)] / plsc.load_gather(denv, [dv + NP])
                plsc.store_scatter(attw, [2 * b + iota * 2], a0)
                plsc.store_scatter(attw, [2 * b + iota * 2 + 1], a1)
                dl = dv - qbase
                mask = (dl >= 0) & (dl < Q)
                cnt = jnp.sum(mask.astype(_i32))
                sv = sidw[pl.ds(b, L)]
                plsc.store_compressed(psrc.at[pl.ds(pc, L)], sv, mask=mask)
                plsc.store_compressed(pdl.at[pl.ds(pc, L)], dl, mask=mask)
                plsc.store_compressed(pw0.at[pl.ds(pc, L)], a0, mask=mask)
                plsc.store_compressed(pw1.at[pl.ds(pc, L)], a1, mask=mask)
                return pc + cnt

            pc = lax.fori_loop(0, WIN // L, scan, jnp.int32(0))

            @pl.when((q == 0) & (c == 0))
            def _():
                pltpu.sync_copy(attw, attn_hbm.at[pl.ds(2 * w0e, 2 * WIN)])

            for t in (0, L):
                pdl[pl.ds(pc + t, L)] = jnp.zeros((L,), _i32)
                pw0[pl.ds(pc + t, L)] = jnp.zeros((L,), _f32)
                pw1[pl.ds(pc + t, L)] = jnp.zeros((L,), _f32)
                psrc[pl.ds(pc + t, L)] = jnp.zeros((L,), _i32)

            ng = (pc + G - 1) // G

            @pl.when(ng > 0)
            def _():
                issue(rowsA, semA, 0)

                def piter(m, _):
                    ka = 2 * m
                    kb = ka + 1

                    @pl.when((kb < ng) & (kb >= 2))
                    def _():
                        wait_s(rowsB, semSB)

                    @pl.when(kb < ng)
                    def _():
                        issue(rowsB, semB, kb * G)

                    wait(rowsA, semA)
                    do_wloop(rowsA, ka)
                    issue_s(rowsA, semSA, ka)

                    @pl.when(kb < ng)
                    def _():
                        wait(rowsB, semB)
                        do_wloop(rowsB, kb)
                        issue_s(rowsB, semSB, kb)

                    @pl.when(ka + 2 < ng)
                    def _():
                        wait_s(rowsA, semSA)
                        issue(rowsA, semA, (ka + 2) * G)
                    return 0

                lax.fori_loop(0, (ng + 1) // 2, piter, 0)
                wait_s(rowsA, semSA)

                @pl.when(ng > 1)
                def _():
                    wait_s(rowsB, semSB)
            return 0

        lax.fori_loop(0, NWIN, window, 0)
        plsc.subcore_barrier()

        nr0 = 79 * (NS - 1)

        @pl.when(s < NS - 1)
        def _():
            pltpu.sync_copy(acc.at[pl.ds(s * 79, 79)],
                            out_hbm.at[pl.ds(qbase + s * 79, 79)])

        @pl.when(s == NS - 1)
        def _():
            pltpu.sync_copy(acc.at[pl.ds(nr0, Q - nr0)],
                            out_hbm.at[pl.ds(qbase + nr0, Q - nr0)])
        plsc.subcore_barrier()
        return 0

    lax.fori_loop(0, NQ, slab, 0)


def _messages(h_v, src, dst, p0, p1, den, zer):
    mesh = plsc.VectorSubcoreMesh(core_axis_name="c", subcore_axis_name="s")
    fn = pl.kernel(
        _k3_body,
        out_type=(
            jax.ShapeDtypeStruct((N, HD), _f32),
            jax.ShapeDtypeStruct((2 * E,), _f32),
        ),
        mesh=mesh,
        compiler_params=_SC_PARAMS,
        scratch_types=[
            pltpu.VMEM((NSEG,), _f32),
            pltpu.VMEM((WIN,), _i32),
            pltpu.VMEM((WIN,), _i32),
            pltpu.VMEM((WIN,), _f32),
            pltpu.VMEM((WIN,), _f32),
            pltpu.VMEM((2 * WIN,), _f32),
            pltpu.VMEM((WIN + 2 * L,), _i32),
            pltpu.VMEM((WIN + 2 * L,), _i32),
            pltpu.VMEM((WIN + 2 * L,), _f32),
            pltpu.VMEM((WIN + 2 * L,), _f32),
            pltpu.VMEM((G, HD), _f32),
            pltpu.VMEM((G, HD), _f32),
            pltpu.VMEM_SHARED((QR, HD), _f32),
            pltpu.SemaphoreType.DMA,
            pltpu.SemaphoreType.DMA,
            pltpu.SemaphoreType.DMA,
            pltpu.SemaphoreType.DMA,
        ],
    )
    return fn(h_v, src, dst, p0, p1, den, zer)


def kernel(x, edge_index, W_src, W_dst, W_val, attn_vec):
    src = edge_index[0].astype(_i32)
    dst = edge_index[1].astype(_i32)
    h_s, h_d, h_v = _projections(x, W_src, W_dst, W_val)
    av = attn_vec.reshape(HD)
    p0, p1, s0, s1 = _edge_scores(h_s, h_d, src, dst, av)
    psum, pmax = _segment_tables(dst, p0, p1, s0, s1)
    den = _denominator(psum, pmax).reshape(NSEG)
    zer = jnp.zeros((QR // NS, HD), _f32)
    out, attn_flat = _messages(h_v, src, dst, p0, p1, den, zer)
    return (out, attn_flat.reshape(E, 2))


# speedup vs baseline: 1.0244x; 1.0244x over previous
import jax
import jax.numpy as jnp
from jax import lax
from jax.experimental import pallas as pl
from jax.experimental.pallas import tpu as pltpu
from jax.experimental.pallas import tpu_sc as plsc

N = 10000
E = 160000
IN_DIM = 256
HD = 512
DH = 256
ALPHA = 0.2

NC = 2
NS = 16
NW = NC * NS
L = 16
G = 32

NP = 10240
NSEG = 2 * NP

EW = 5024
EW_LAST = E - (NW - 1) * EW
ES = E // NS
WIN = 2000
NWIN = ES // WIN

Q = 1250
QR = 1280
NQ = 4

_f32 = jnp.float32
_i32 = jnp.int32

_SC_PARAMS = pltpu.CompilerParams(
    use_tc_tiling_on_sc=False, needs_layout_passes=False)


def _mm_body(x_ref, ws_ref, wd_ref, wv_ref, hs_ref, hd_ref, hv_ref):
    xb = x_ref[...]
    hs_ref[...] = jnp.dot(xb, ws_ref[...], preferred_element_type=_f32)
    hd_ref[...] = jnp.dot(xb, wd_ref[...], preferred_element_type=_f32)
    hv_ref[...] = jnp.dot(xb, wv_ref[...], preferred_element_type=_f32)


def _projections(x, w_s, w_d, w_v):
    blk = 1000
    return pl.pallas_call(
        _mm_body,
        grid=(N // blk,),
        in_specs=[
            pl.BlockSpec((blk, IN_DIM), lambda i: (i, 0)),
            pl.BlockSpec((IN_DIM, HD), lambda i: (0, 0)),
            pl.BlockSpec((IN_DIM, HD), lambda i: (0, 0)),
            pl.BlockSpec((IN_DIM, HD), lambda i: (0, 0)),
        ],
        out_specs=[
            pl.BlockSpec((blk, HD), lambda i: (i, 0)),
            pl.BlockSpec((blk, HD), lambda i: (i, 0)),
            pl.BlockSpec((blk, HD), lambda i: (i, 0)),
        ],
        out_shape=[
            jax.ShapeDtypeStruct((N, HD), _f32),
            jax.ShapeDtypeStruct((N, HD), _f32),
            jax.ShapeDtypeStruct((N, HD), _f32),
        ],
    )(x, w_s, w_d, w_v)


def _k2_body(hs_hbm, hd_hbm, src_hbm, dst_hbm, av_hbm,
             p0_hbm, p1_hbm, s0_hbm, s1_hbm,
             srca, dsta, avv, p0a, p1a, s0a, s1a,
             hsA, hdA, hsB, hdB, semA1, semA2, semB1, semB2):
    c = lax.axis_index("c")
    s = lax.axis_index("s")
    wid = s * NC + c
    base = wid * EW
    last = wid == NW - 1
    ng = jnp.where(last, EW_LAST // G, EW // G)

    pltpu.sync_copy(av_hbm, avv)

    @pl.when(jnp.logical_not(last))
    def _():
        pltpu.sync_copy(src_hbm.at[pl.ds(base, EW)], srca.at[pl.ds(0, EW)])
        pltpu.sync_copy(dst_hbm.at[pl.ds(base, EW)], dsta.at[pl.ds(0, EW)])

    @pl.when(last)
    def _():
        pltpu.sync_copy(src_hbm.at[pl.ds(base, EW_LAST)],
                        srca.at[pl.ds(0, EW_LAST)])
        pltpu.sync_copy(dst_hbm.at[pl.ds(base, EW_LAST)],
                        dsta.at[pl.ds(0, EW_LAST)])

    iota = lax.iota(_i32, L)

    def issue(hsX, hdX, semX1, semX2, k):
        b = k * G
        pltpu.async_copy(hs_hbm.at[srca.at[pl.ds(b, G)]], hsX, semX1)
        pltpu.async_copy(hd_hbm.at[dsta.at[pl.ds(b, G)]], hdX, semX2)

    def wait(hsX, hdX, semX1, semX2):
        pltpu.make_async_copy(
            hs_hbm.at[srca.at[pl.ds(0, G)]], hsX, semX1).wait()
        pltpu.make_async_copy(
            hd_hbm.at[dsta.at[pl.ds(0, G)]], hdX, semX2).wait()

    def do_group(hsX, hdX, k):
        b = k * G
        for h in (0, 1):
            rid = iota + L * h

            def dloop(j, accs):
                acc0, acc1 = accs
                for u in range(4):
                    d = j * 4 + u
                    df = ((d + iota) & (L - 1)) + (d & ~(L - 1))
                    z0 = (plsc.load_gather(hsX, [rid, df])
                          + plsc.load_gather(hdX, [rid, df]))
                    l0 = jnp.maximum(z0, 0.0) + ALPHA * jnp.minimum(z0, 0.0)
                    acc0 = acc0 + l0 * plsc.load_gather(avv, [df])
                    df1 = df + DH
                    z1 = (plsc.load_gather(hsX, [rid, df1])
                          + plsc.load_gather(hdX, [rid, df1]))
                    l1 = jnp.maximum(z1, 0.0) + ALPHA * jnp.minimum(z1, 0.0)
                    acc1 = acc1 + l1 * plsc.load_gather(avv, [df1])
                return (acc0, acc1)

            acc0, acc1 = lax.fori_loop(
                0, DH // 4, dloop,
                (jnp.zeros((L,), _f32), jnp.zeros((L,), _f32)))
            o = b + L * h
            s0a[pl.ds(o, L)] = acc0
            s1a[pl.ds(o, L)] = acc1
            p0a[pl.ds(o, L)] = jnp.exp(acc0)
            p1a[pl.ds(o, L)] = jnp.exp(acc1)

    issue(hsA, hdA, semA1, semA2, 0)

    def piter(m, _):
        ka = 2 * m
        kb = ka + 1

        @pl.when(kb < ng)
        def _():
            issue(hsB, hdB, semB1, semB2, kb)

        wait(hsA, hdA, semA1, semA2)
        do_group(hsA, hdA, ka)

        @pl.when(ka + 2 < ng)
        def _():
            issue(hsA, hdA, semA1, semA2, ka + 2)

        @pl.when(kb < ng)
        def _():
            wait(hsB, hdB, semB1, semB2)
            do_group(hsB, hdB, kb)
        return 0

    lax.fori_loop(0, (ng + 1) // 2, piter, 0)

    @pl.when(jnp.logical_not(last))
    def _():
        pltpu.sync_copy(p0a.at[pl.ds(0, EW)], p0_hbm.at[pl.ds(base, EW)])
        pltpu.sync_copy(p1a.at[pl.ds(0, EW)], p1_hbm.at[pl.ds(base, EW)])
        pltpu.sync_copy(s0a.at[pl.ds(0, EW)], s0_hbm.at[pl.ds(base, EW)])
        pltpu.sync_copy(s1a.at[pl.ds(0, EW)], s1_hbm.at[pl.ds(base, EW)])

    @pl.when(last)
    def _():
        pltpu.sync_copy(p0a.at[pl.ds(0, EW_LAST)],
                        p0_hbm.at[pl.ds(base, EW_LAST)])
        pltpu.sync_copy(p1a.at[pl.ds(0, EW_LAST)],
                        p1_hbm.at[pl.ds(base, EW_LAST)])
        pltpu.sync_copy(s0a.at[pl.ds(0, EW_LAST)],
                        s0_hbm.at[pl.ds(base, EW_LAST)])
        pltpu.sync_copy(s1a.at[pl.ds(0, EW_LAST)],
                        s1_hbm.at[pl.ds(base, EW_LAST)])


def _edge_scores(h_s, h_d, src, dst, av):
    mesh = plsc.VectorSubcoreMesh(core_axis_name="c", subcore_axis_name="s")
    fn = pl.kernel(
        _k2_body,
        out_type=(
            jax.ShapeDtypeStruct((E,), _f32),
            jax.ShapeDtypeStruct((E,), _f32),
            jax.ShapeDtypeStruct((E,), _f32),
            jax.ShapeDtypeStruct((E,), _f32),
        ),
        mesh=mesh,
        compiler_params=_SC_PARAMS,
        scratch_types=[
            pltpu.VMEM((EW,), _i32),
            pltpu.VMEM((EW,), _i32),
            pltpu.VMEM((HD,), _f32),
            pltpu.VMEM((EW,), _f32),
            pltpu.VMEM((EW,), _f32),
            pltpu.VMEM((EW,), _f32),
            pltpu.VMEM((EW,), _f32),
            pltpu.VMEM((G, HD), _f32),
            pltpu.VMEM((G, HD), _f32),
            pltpu.VMEM((G, HD), _f32),
            pltpu.VMEM((G, HD), _f32),
            pltpu.SemaphoreType.DMA,
            pltpu.SemaphoreType.DMA,
            pltpu.SemaphoreType.DMA,
            pltpu.SemaphoreType.DMA,
        ],
    )
    return fn(h_s, h_d, src, dst, av)


def _kseg_body(dst_hbm, p0_hbm, p1_hbm, s0_hbm, s1_hbm,
               psum_hbm, pmax_hbm,
               dsta, p0a, p1a, s0a, s1a, ssum, smax, widt):
    c = lax.axis_index("c")
    s = lax.axis_index("s")
    wid = s * NC + c
    base = wid * EW
    last = wid == NW - 1
    ng = jnp.where(last, EW_LAST // L, EW // L)

    @pl.when(jnp.logical_not(last))
    def _():
        pltpu.sync_copy(dst_hbm.at[pl.ds(base, EW)], dsta.at[pl.ds(0, EW)])
        pltpu.sync_copy(p0_hbm.at[pl.ds(base, EW)], p0a.at[pl.ds(0, EW)])
        pltpu.sync_copy(p1_hbm.at[pl.ds(base, EW)], p1a.at[pl.ds(0, EW)])
        pltpu.sync_copy(s0_hbm.at[pl.ds(base, EW)], s0a.at[pl.ds(0, EW)])
        pltpu.sync_copy(s1_hbm.at[pl.ds(base, EW)], s1a.at[pl.ds(0, EW)])

    @pl.when(last)
    def _():
        pltpu.sync_copy(dst_hbm.at[pl.ds(base, EW_LAST)],
                        dsta.at[pl.ds(0, EW_LAST)])
        pltpu.sync_copy(p0_hbm.at[pl.ds(base, EW_LAST)],
                        p0a.at[pl.ds(0, EW_LAST)])
        pltpu.sync_copy(p1_hbm.at[pl.ds(base, EW_LAST)],
                        p1a.at[pl.ds(0, EW_LAST)])
        pltpu.sync_copy(s0_hbm.at[pl.ds(base, EW_LAST)],
                        s0a.at[pl.ds(0, EW_LAST)])
        pltpu.sync_copy(s1_hbm.at[pl.ds(base, EW_LAST)],
                        s1a.at[pl.ds(0, EW_LAST)])

    def init_body(i, _):
        ssum[pl.ds(i * L, L)] = jnp.zeros((L,), _f32)
        smax[pl.ds(i * L, L)] = jnp.full((L,), -1e30, _f32)
        return 0
    lax.fori_loop(0, NSEG // L, init_body, 0)

    iota = lax.iota(_i32, L)

    def group(g, _):
        b = g * L
        dv = dsta[pl.ds(b, L)]
        dv1 = dv + NP
        sc0 = s0a[pl.ds(b, L)]
        sc1 = s1a[pl.ds(b, L)]
        p0v = p0a[pl.ds(b, L)]
        p1v = p1a[pl.ds(b, L)]

        def upd_cond(rem):
            return jnp.any(rem)

        def upd_body(rem):
            plsc.store_scatter(widt, [dv], iota, mask=rem)
            win = rem & (plsc.load_gather(widt, [dv]) == iota)
            plsc.store_scatter(
                smax, [dv],
                jnp.maximum(plsc.load_gather(smax, [dv]), sc0), mask=win)
            plsc.store_scatter(
                smax, [dv1],
                jnp.maximum(plsc.load_gather(smax, [dv1]), sc1), mask=win)
            plsc.store_scatter(
                ssum, [dv], plsc.load_gather(ssum, [dv]) + p0v, mask=win)
            plsc.store_scatter(
                ssum, [dv1], plsc.load_gather(ssum, [dv1]) + p1v, mask=win)
            return rem & jnp.logical_not(win)

        lax.while_loop(upd_cond, upd_body, jnp.full((L,), True, jnp.bool_))
        return 0

    lax.fori_loop(0, ng, group, 0)
    pltpu.sync_copy(ssum, psum_hbm.at[pl.ds(wid * NSEG, NSEG)])
    pltpu.sync_copy(smax, pmax_hbm.at[pl.ds(wid * NSEG, NSEG)])


def _segment_tables(dst, p0, p1, s0, s1):
    mesh = plsc.VectorSubcoreMesh(core_axis_name="c", subcore_axis_name="s")
    fn = pl.kernel(
        _kseg_body,
        out_type=(
            jax.ShapeDtypeStruct((NW * NSEG,), _f32),
            jax.ShapeDtypeStruct((NW * NSEG,), _f32),
        ),
        mesh=mesh,
        compiler_params=_SC_PARAMS,
        scratch_types=[
            pltpu.VMEM((EW,), _i32),
            pltpu.VMEM((EW,), _f32),
            pltpu.VMEM((EW,), _f32),
            pltpu.VMEM((EW,), _f32),
            pltpu.VMEM((EW,), _f32),
            pltpu.VMEM((NSEG,), _f32),
            pltpu.VMEM((NSEG,), _f32),
            pltpu.VMEM((NP,), _i32),
        ],
    )
    return fn(dst, p0, p1, s0, s1)


def _k2b_body(ps_ref, pm_ref, den_ref):
    ssum = jnp.sum(ps_ref[...], axis=0, keepdims=True)
    smax = jnp.max(pm_ref[...], axis=0, keepdims=True)
    den_ref[...] = ssum + 1e-9 * jnp.exp(smax)


def _denominator(psum, pmax):
    return pl.pallas_call(
        _k2b_body,
        out_shape=jax.ShapeDtypeStruct((1, NSEG), _f32),
    )(psum.reshape(NW, NSEG), pmax.reshape(NW, NSEG))


def _k3_body(hv_hbm, src_hbm, dst_hbm, p0_hbm, p1_hbm, den_hbm, zer_hbm,
             out_hbm, attn_hbm,
             denv, sidw, didw, p0w, p1w, attw,
             psrc, pdl, pw0, pw1, rowsA, rowsB, acc,
             semA, semB, semSA, semSB):
    c = lax.axis_index("c")
    s = lax.axis_index("s")
    ebase = s * ES
    iota = lax.iota(_i32, L)

    pltpu.sync_copy(den_hbm, denv)

    def initp(i, _):
        psrc[pl.ds(i * L, L)] = jnp.zeros((L,), _i32)
        return 0
    lax.fori_loop(0, (WIN + 2 * L) // L, initp, 0)

    def issue(rowsX, semX, b):
        pltpu.async_copy(hv_hbm.at[psrc.at[pl.ds(b, G)]], rowsX, semX)

    def wait(rowsX, semX):
        pltpu.make_async_copy(
            hv_hbm.at[psrc.at[pl.ds(0, G)]], rowsX, semX).wait()

    def do_wloop(rowsX, k):
        b = k * G
        for h in (0, 1):
            rid = iota + L * h
            w0 = pw0[pl.ds(b + L * h, L)]
            w1 = pw1[pl.ds(b + L * h, L)]

            def wloop(j, _):
                for u in range(4):
                    d = j * 4 + u
                    df = ((d + iota) & (L - 1)) + (d & ~(L - 1))
                    plsc.store_scatter(
                        rowsX, [rid, df],
                        plsc.load_gather(rowsX, [rid, df]) * w0)
                    df1 = df + DH
                    plsc.store_scatter(
                        rowsX, [rid, df1],
                        plsc.load_gather(rowsX, [rid, df1]) * w1)
                return 0
            lax.fori_loop(0, DH // 4, wloop, 0)

    def issue_s(rowsX, semSX, k):
        b = k * G
        dlv0 = pdl[pl.ds(b, L)]
        dlv1 = pdl[pl.ds(b + L, L)]
        pltpu.async_copy(rowsX.at[pl.ds(0, L)], acc.at[dlv0], semSX,
                         add=True)
        pltpu.async_copy(rowsX.at[pl.ds(L, L)], acc.at[dlv1], semSX,
                         add=True)

    def wait_s(rowsX, semSX):
        pltpu.make_async_copy(
            rowsX.at[pl.ds(0, L)], acc.at[iota], semSX).wait()
        pltpu.make_async_copy(
            rowsX.at[pl.ds(L, L)], acc.at[iota], semSX).wait()

    def slab(q, _):
        qbase = c * (NQ * Q) + q * Q

        pltpu.sync_copy(zer_hbm, acc.at[pl.ds(s * (QR // NS), QR // NS)])
        plsc.subcore_barrier()

        def window(w, _):
            w0e = ebase + w * WIN
            pltpu.sync_copy(src_hbm.at[pl.ds(w0e, WIN)], sidw)
            pltpu.sync_copy(dst_hbm.at[pl.ds(w0e, WIN)], didw)
            pltpu.sync_copy(p0_hbm.at[pl.ds(w0e, WIN)], p0w)
            pltpu.sync_copy(p1_hbm.at[pl.ds(w0e, WIN)], p1w)

            def scan(g, pc):
                b = g * L
                dv = didw[pl.ds(b, L)]
                a0 = p0w[pl.ds(b, L)] / plsc.load_gather(denv, [dv])
                a1 = p1w[pl.ds(b, L)] / plsc.load_gather(denv, [dv + NP])
                plsc.store_scatter(attw, [2 * b + iota * 2], a0)
                plsc.store_scatter(attw, [2 * b + iota * 2 + 1], a1)
                dl = dv - qbase
                mask = (dl >= 0) & (dl < Q)
                cnt = jnp.sum(mask.astype(_i32))
                sv = sidw[pl.ds(b, L)]
                plsc.store_compressed(psrc.at[pl.ds(pc, L)], sv, mask=mask)
                plsc.store_compressed(pdl.at[pl.ds(pc, L)], dl, mask=mask)
                plsc.store_compressed(pw0.at[pl.ds(pc, L)], a0, mask=mask)
                plsc.store_compressed(pw1.at[pl.ds(pc, L)], a1, mask=mask)
                return pc + cnt

            pc = lax.fori_loop(0, WIN // L, scan, jnp.int32(0))

            @pl.when((q == 0) & (c == 0))
            def _():
                pltpu.sync_copy(attw, attn_hbm.at[pl.ds(2 * w0e, 2 * WIN)])

            for t in (0, L):
                pdl[pl.ds(pc + t, L)] = jnp.zeros((L,), _i32)
                pw0[pl.ds(pc + t, L)] = jnp.zeros((L,), _f32)
                pw1[pl.ds(pc + t, L)] = jnp.zeros((L,), _f32)
                psrc[pl.ds(pc + t, L)] = jnp.zeros((L,), _i32)

            ng = (pc + G - 1) // G

            @pl.when(ng > 0)
            def _():
                issue(rowsA, semA, 0)

                def piter(m, _):
                    ka = 2 * m
                    kb = ka + 1

                    @pl.when((kb < ng) & (kb >= 2))
                    def _():
                        wait_s(rowsB, semSB)

                    @pl.when(kb < ng)
                    def _():
                        issue(rowsB, semB, kb * G)

                    wait(rowsA, semA)
                    do_wloop(rowsA, ka)
                    issue_s(rowsA, semSA, ka)

                    @pl.when(kb < ng)
                    def _():
                        wait(rowsB, semB)
                        do_wloop(rowsB, kb)
                        issue_s(rowsB, semSB, kb)

                    @pl.when(ka + 2 < ng)
                    def _():
                        wait_s(rowsA, semSA)
                        issue(rowsA, semA, (ka + 2) * G)
                    return 0

                lax.fori_loop(0, (ng + 1) // 2, piter, 0)
                wait_s(rowsA, semSA)

                @pl.when(ng > 1)
                def _():
                    wait_s(rowsB, semSB)
            return 0

        lax.fori_loop(0, NWIN, window, 0)
        plsc.subcore_barrier()

        nr0 = 79 * (NS - 1)

        @pl.when(s < NS - 1)
        def _():
            pltpu.sync_copy(acc.at[pl.ds(s * 79, 79)],
                            out_hbm.at[pl.ds(qbase + s * 79, 79)])

        @pl.when(s == NS - 1)
        def _():
            pltpu.sync_copy(acc.at[pl.ds(nr0, Q - nr0)],
                            out_hbm.at[pl.ds(qbase + nr0, Q - nr0)])
        plsc.subcore_barrier()
        return 0

    lax.fori_loop(0, NQ, slab, 0)


def _messages(h_v, src, dst, p0, p1, den, zer):
    mesh = plsc.VectorSubcoreMesh(core_axis_name="c", subcore_axis_name="s")
    fn = pl.kernel(
        _k3_body,
        out_type=(
            jax.ShapeDtypeStruct((N, HD), _f32),
            jax.ShapeDtypeStruct((2 * E,), _f32),
        ),
        mesh=mesh,
        compiler_params=_SC_PARAMS,
        scratch_types=[
            pltpu.VMEM((NSEG,), _f32),
            pltpu.VMEM((WIN,), _i32),
            pltpu.VMEM((WIN,), _i32),
            pltpu.VMEM((WIN,), _f32),
            pltpu.VMEM((WIN,), _f32),
            pltpu.VMEM((2 * WIN,), _f32),
            pltpu.VMEM((WIN + 2 * L,), _i32),
            pltpu.VMEM((WIN + 2 * L,), _i32),
            pltpu.VMEM((WIN + 2 * L,), _f32),
            pltpu.VMEM((WIN + 2 * L,), _f32),
            pltpu.VMEM((G, HD), _f32),
            pltpu.VMEM((G, HD), _f32),
            pltpu.VMEM_SHARED((QR, HD), _f32),
            pltpu.SemaphoreType.DMA,
            pltpu.SemaphoreType.DMA,
            pltpu.SemaphoreType.DMA,
            pltpu.SemaphoreType.DMA,
        ],
    )
    return fn(h_v, src, dst, p0, p1, den, zer)


def kernel(x, edge_index, W_src, W_dst, W_val, attn_vec):
    src = edge_index[0].astype(_i32)
    dst = edge_index[1].astype(_i32)
    h_s, h_d, h_v = _projections(x, W_src, W_dst, W_val)
    av = attn_vec.reshape(HD)
    p0, p1, s0, s1 = _edge_scores(h_s, h_d, src, dst, av)
    psum, pmax = _segment_tables(dst, p0, p1, s0, s1)
    den = _denominator(psum, pmax).reshape(NSEG)
    zer = jnp.zeros((QR // NS, HD), _f32)
    out, attn_flat = _messages(h_v, src, dst, p0, p1, den, zer)
    return (out, attn_flat.reshape(E, 2))
